# initial kernel scaffold (unmeasured)
import jax
import jax.numpy as jnp
from jax import lax
from jax.experimental import pallas as pl
from jax.experimental.pallas import tpu as pltpu

N_DEV = 16


def kernel(x, w_mat, scale_x, scale_w):
    M, K = x.shape
    N = w_mat.shape[1]
    CH = M // N_DEV

    def body(x_ref, w_ref, sx_ref, sw_ref, out_ref,
             comm1, send1, recv1, credit1,
             comm2, send2, recv2, credit2):
        p = lax.axis_index("i")
        left = lax.rem(p + N_DEV - 1, N_DEV)
        right = lax.rem(p + 1, N_DEV)

        out_ref[...] = jnp.dot(x_ref[...], w_ref[...],
                               preferred_element_type=jnp.float32)

        for t in range(N_DEV - 1):
            slot = t % 2
            if t >= 2:
                pl.semaphore_wait(credit1, 1)
            send_c = lax.rem(p - t + 2 * N_DEV, N_DEV)
            rdma = pltpu.make_async_remote_copy(
                src_ref=out_ref.at[pl.ds(send_c * CH, CH)],
                dst_ref=comm1.at[slot],
                send_sem=send1.at[slot],
                recv_sem=recv1.at[slot],
                device_id=(right,),
                device_id_type=pl.DeviceIdType.MESH,
            )
            rdma.start()
            rdma.wait()
            add_c = lax.rem(p - t - 1 + 2 * N_DEV, N_DEV)
            out_ref[pl.ds(add_c * CH, CH)] = (
                out_ref[pl.ds(add_c * CH, CH)] + comm1[slot])
            if t <= N_DEV - 4:
                pl.semaphore_signal(credit1, inc=1, device_id=(left,),
                                    device_id_type=pl.DeviceIdType.MESH)

        for t in range(N_DEV - 1):
            slot = t % 2
            if t >= 2:
                pl.semaphore_wait(credit2, 1)
            src_c = lax.rem(p + 1 - t + 2 * N_DEV, N_DEV)
            rdma = pltpu.make_async_remote_copy(
                src_ref=out_ref.at[pl.ds(src_c * CH, CH)],
                dst_ref=comm2.at[slot],
                send_sem=send2.at[slot],
                recv_sem=recv2.at[slot],
                device_id=(right,),
                device_id_type=pl.DeviceIdType.MESH,
            )
            rdma.start()
            rdma.wait()
            dst_c = lax.rem(p - t + 2 * N_DEV, N_DEV)
            out_ref[pl.ds(dst_c * CH, CH)] = comm2[slot]
            if t <= N_DEV - 4:
                pl.semaphore_signal(credit2, inc=1, device_id=(left,),
                                    device_id_type=pl.DeviceIdType.MESH)

        s = sx_ref[0] * sw_ref[0]
        y = out_ref[...] * s
        out_ref[...] = y * jax.nn.sigmoid(y)

    return pl.pallas_call(
        body,
        out_shape=jax.ShapeDtypeStruct((M, N), jnp.float32),
        in_specs=[
            pl.BlockSpec(memory_space=pltpu.VMEM),
            pl.BlockSpec(memory_space=pltpu.VMEM),
            pl.BlockSpec(memory_space=pltpu.SMEM),
            pl.BlockSpec(memory_space=pltpu.SMEM),
        ],
        out_specs=pl.BlockSpec(memory_space=pltpu.VMEM),
        scratch_shapes=[
            pltpu.VMEM((2, CH, N), jnp.float32),
            pltpu.SemaphoreType.DMA((2,)),
            pltpu.SemaphoreType.DMA((2,)),
            pltpu.SemaphoreType.REGULAR,
            pltpu.VMEM((2, CH, N), jnp.float32),
            pltpu.SemaphoreType.DMA((2,)),
            pltpu.SemaphoreType.DMA((2,)),
            pltpu.SemaphoreType.REGULAR,
        ],
    )(x, w_mat, scale_x, scale_w)


# baseline (device time: 797986 ns/iter reference)
import jax
import jax.numpy as jnp
from jax import lax
from jax.experimental import pallas as pl
from jax.experimental.pallas import tpu as pltpu

N_DEV = 16


def kernel(x, w_mat, scale_x, scale_w):
    M, K = x.shape
    N = w_mat.shape[1]
    CH = M // N_DEV

    def body(x_ref, w_ref, sx_ref, sw_ref, out_ref,
             comm1, send1, recv1, credit1,
             comm2, send2, recv2, credit2):
        p = lax.axis_index("i")
        left = lax.rem(p + N_DEV - 1, N_DEV)
        right = lax.rem(p + 1, N_DEV)

        w = w_ref[...].astype(jnp.bfloat16)
        for c in range(N_DEV):
            out_ref[c * CH:(c + 1) * CH] = jnp.dot(
                x_ref[c * CH:(c + 1) * CH].astype(jnp.bfloat16), w,
                preferred_element_type=jnp.float32)

        for t in range(N_DEV - 1):
            slot = t % 2
            if t >= 2:
                pl.semaphore_wait(credit1, 1)
            send_c = lax.rem(p - t + 2 * N_DEV, N_DEV)
            rdma = pltpu.make_async_remote_copy(
                src_ref=out_ref.at[pl.ds(send_c * CH, CH)],
                dst_ref=comm1.at[slot],
                send_sem=send1.at[slot],
                recv_sem=recv1.at[slot],
                device_id=(right,),
                device_id_type=pl.DeviceIdType.MESH,
            )
            rdma.start()
            rdma.wait()
            add_c = lax.rem(p - t - 1 + 2 * N_DEV, N_DEV)
            out_ref[pl.ds(add_c * CH, CH)] = (
                out_ref[pl.ds(add_c * CH, CH)] + comm1[slot])
            if t <= N_DEV - 4:
                pl.semaphore_signal(credit1, inc=1, device_id=(left,),
                                    device_id_type=pl.DeviceIdType.MESH)

        for t in range(N_DEV - 1):
            slot = t % 2
            if t >= 2:
                pl.semaphore_wait(credit2, 1)
            src_c = lax.rem(p + 1 - t + 2 * N_DEV, N_DEV)
            rdma = pltpu.make_async_remote_copy(
                src_ref=out_ref.at[pl.ds(src_c * CH, CH)],
                dst_ref=comm2.at[slot],
                send_sem=send2.at[slot],
                recv_sem=recv2.at[slot],
                device_id=(right,),
                device_id_type=pl.DeviceIdType.MESH,
            )
            rdma.start()
            rdma.wait()
            dst_c = lax.rem(p - t + 2 * N_DEV, N_DEV)
            out_ref[pl.ds(dst_c * CH, CH)] = comm2[slot]
            if t <= N_DEV - 4:
                pl.semaphore_signal(credit2, inc=1, device_id=(left,),
                                    device_id_type=pl.DeviceIdType.MESH)

        s = sx_ref[0] * sw_ref[0]
        for c in range(N_DEV):
            y = out_ref[c * CH:(c + 1) * CH] * s
            out_ref[c * CH:(c + 1) * CH] = y * jax.nn.sigmoid(y)

    return pl.pallas_call(
        body,
        out_shape=jax.ShapeDtypeStruct((M, N), jnp.float32),
        in_specs=[
            pl.BlockSpec(memory_space=pltpu.VMEM),
            pl.BlockSpec(memory_space=pltpu.VMEM),
            pl.BlockSpec(memory_space=pltpu.SMEM),
            pl.BlockSpec(memory_space=pltpu.SMEM),
        ],
        out_specs=pl.BlockSpec(memory_space=pltpu.VMEM),
        scratch_shapes=[
            pltpu.VMEM((2, CH, N), jnp.float32),
            pltpu.SemaphoreType.DMA((2,)),
            pltpu.SemaphoreType.DMA((2,)),
            pltpu.SemaphoreType.REGULAR,
            pltpu.VMEM((2, CH, N), jnp.float32),
            pltpu.SemaphoreType.DMA((2,)),
            pltpu.SemaphoreType.DMA((2,)),
            pltpu.SemaphoreType.REGULAR,
        ],
        compiler_params=pltpu.CompilerParams(
            vmem_limit_bytes=56 * 1024 * 1024),
    )(x, w_mat, scale_x, scale_w)


# device time: 463083 ns/iter; 1.7232x vs baseline; 1.7232x over previous
import jax
import jax.numpy as jnp
from jax import lax
from jax.experimental import pallas as pl
from jax.experimental.pallas import tpu as pltpu

N_DEV = 16


def kernel(x, w_mat, scale_x, scale_w):
    M, K = x.shape
    N = w_mat.shape[1]
    CH = M // N_DEV

    def body(x_ref, w_ref, sx_ref, sw_ref, out_ref,
             sbuf1, comm1, send1, recv1, credit1,
             sbuf2, comm2, send2, recv2, credit2):
        p = lax.axis_index("i")
        left = lax.rem(p + N_DEV - 1, N_DEV)
        right = lax.rem(p + 1, N_DEV)

        w = w_ref[...].astype(jnp.bfloat16)
        for c in range(N_DEV):
            out_ref[c * CH:(c + 1) * CH] = jnp.dot(
                x_ref[c * CH:(c + 1) * CH].astype(jnp.bfloat16), w,
                preferred_element_type=jnp.float32)

        for t in range(N_DEV - 1):
            slot = t % 2
            if t >= 2:
                pl.semaphore_wait(credit1, 1)
            send_c = lax.rem(p - t + 2 * N_DEV, N_DEV)
            sbuf1[slot] = out_ref[pl.ds(send_c * CH, CH)].astype(jnp.bfloat16)
            rdma = pltpu.make_async_remote_copy(
                src_ref=sbuf1.at[slot],
                dst_ref=comm1.at[slot],
                send_sem=send1.at[slot],
                recv_sem=recv1.at[slot],
                device_id=(right,),
                device_id_type=pl.DeviceIdType.MESH,
            )
            rdma.start()
            rdma.wait()
            add_c = lax.rem(p - t - 1 + 2 * N_DEV, N_DEV)
            out_ref[pl.ds(add_c * CH, CH)] = (
                out_ref[pl.ds(add_c * CH, CH)]
                + comm1[slot].astype(jnp.float32))
            if t <= N_DEV - 4:
                pl.semaphore_signal(credit1, inc=1, device_id=(left,),
                                    device_id_type=pl.DeviceIdType.MESH)

        for t in range(N_DEV - 1):
            slot = t % 2
            if t >= 2:
                pl.semaphore_wait(credit2, 1)
            src_c = lax.rem(p + 1 - t + 2 * N_DEV, N_DEV)
            sbuf2[slot] = out_ref[pl.ds(src_c * CH, CH)].astype(jnp.bfloat16)
            rdma = pltpu.make_async_remote_copy(
                src_ref=sbuf2.at[slot],
                dst_ref=comm2.at[slot],
                send_sem=send2.at[slot],
                recv_sem=recv2.at[slot],
                device_id=(right,),
                device_id_type=pl.DeviceIdType.MESH,
            )
            rdma.start()
            rdma.wait()
            dst_c = lax.rem(p - t + 2 * N_DEV, N_DEV)
            out_ref[pl.ds(dst_c * CH, CH)] = comm2[slot].astype(jnp.float32)
            if t <= N_DEV - 4:
                pl.semaphore_signal(credit2, inc=1, device_id=(left,),
                                    device_id_type=pl.DeviceIdType.MESH)

        s = sx_ref[0] * sw_ref[0]
        for c in range(N_DEV):
            y = out_ref[c * CH:(c + 1) * CH] * s
            out_ref[c * CH:(c + 1) * CH] = y * jax.nn.sigmoid(y)

    return pl.pallas_call(
        body,
        out_shape=jax.ShapeDtypeStruct((M, N), jnp.float32),
        in_specs=[
            pl.BlockSpec(memory_space=pltpu.VMEM),
            pl.BlockSpec(memory_space=pltpu.VMEM),
            pl.BlockSpec(memory_space=pltpu.SMEM),
            pl.BlockSpec(memory_space=pltpu.SMEM),
        ],
        out_specs=pl.BlockSpec(memory_space=pltpu.VMEM),
        scratch_shapes=[
            pltpu.VMEM((2, CH, N), jnp.bfloat16),
            pltpu.VMEM((2, CH, N), jnp.bfloat16),
            pltpu.SemaphoreType.DMA((2,)),
            pltpu.SemaphoreType.DMA((2,)),
            pltpu.SemaphoreType.REGULAR,
            pltpu.VMEM((2, CH, N), jnp.bfloat16),
            pltpu.VMEM((2, CH, N), jnp.bfloat16),
            pltpu.SemaphoreType.DMA((2,)),
            pltpu.SemaphoreType.DMA((2,)),
            pltpu.SemaphoreType.REGULAR,
        ],
        compiler_params=pltpu.CompilerParams(
            vmem_limit_bytes=56 * 1024 * 1024),
    )(x, w_mat, scale_x, scale_w)


# device time: 333263 ns/iter; 2.3945x vs baseline; 1.3895x over previous
import jax
import jax.numpy as jnp
from jax import lax
from jax.experimental import pallas as pl
from jax.experimental.pallas import tpu as pltpu

N_DEV = 16


def kernel(x, w_mat, scale_x, scale_w):
    M, K = x.shape
    N = w_mat.shape[1]
    CH = M // N_DEV
    HN = N // 2
    HOPS = N_DEV - 1

    def body(x_ref, w_ref, sx_ref, sw_ref, out_ref,
             sbufR, commR, sendR, recvR, creditR,
             sbufL, commL, sendL, recvL, creditL):
        p = lax.axis_index("i")
        left = lax.rem(p + N_DEV - 1, N_DEV)
        right = lax.rem(p + 1, N_DEV)

        def row(c):
            return pl.ds(lax.rem(c + 2 * N_DEV, N_DEV) * CH, CH)

        w = w_ref[...].astype(jnp.bfloat16)

        def gemm_step(c, _):
            out_ref[pl.ds(c * CH, CH)] = jnp.dot(
                x_ref[pl.ds(c * CH, CH)].astype(jnp.bfloat16), w,
                preferred_element_type=jnp.float32)
            return _
        lax.fori_loop(0, N_DEV, gemm_step, None)

        def exchange(g, srcR_c, srcL_c):
            slot = lax.rem(g, 2)

            @pl.when(g >= 2)
            def _():
                pl.semaphore_wait(creditR, 1)
                pl.semaphore_wait(creditL, 1)

            sbufR[slot] = out_ref[row(srcR_c), :HN].astype(jnp.bfloat16)
            sbufL[slot] = out_ref[row(srcL_c), HN:].astype(jnp.bfloat16)
            rdmaR = pltpu.make_async_remote_copy(
                src_ref=sbufR.at[slot], dst_ref=commR.at[slot],
                send_sem=sendR.at[slot], recv_sem=recvR.at[slot],
                device_id=(right,), device_id_type=pl.DeviceIdType.MESH)
            rdmaL = pltpu.make_async_remote_copy(
                src_ref=sbufL.at[slot], dst_ref=commL.at[slot],
                send_sem=sendL.at[slot], recv_sem=recvL.at[slot],
                device_id=(left,), device_id_type=pl.DeviceIdType.MESH)
            rdmaR.start()
            rdmaL.start()
            rdmaR.wait()
            rdmaL.wait()
            return slot

        def credits(g):
            @pl.when(g <= 2 * HOPS - 3)
            def _():
                pl.semaphore_signal(creditR, inc=1, device_id=(left,),
                                    device_id_type=pl.DeviceIdType.MESH)
                pl.semaphore_signal(creditL, inc=1, device_id=(right,),
                                    device_id_type=pl.DeviceIdType.MESH)

        def rs_step(t, _):
            slot = exchange(t, p - t, p + t)
            out_ref[row(p - t - 1), :HN] = (
                out_ref[row(p - t - 1), :HN] + commR[slot].astype(jnp.float32))
            out_ref[row(p + t + 1), HN:] = (
                out_ref[row(p + t + 1), HN:] + commL[slot].astype(jnp.float32))
            credits(t)
            return _
        lax.fori_loop(0, HOPS, rs_step, None)

        def ag_step(t, _):
            g = HOPS + t
            slot = exchange(g, p + 1 - t, p - 1 + t)
            out_ref[row(p - t), :HN] = commR[slot].astype(jnp.float32)
            out_ref[row(p + t), HN:] = commL[slot].astype(jnp.float32)
            credits(g)
            return _
        lax.fori_loop(0, HOPS, ag_step, None)

        s = sx_ref[0] * sw_ref[0]

        def epi_step(c, _):
            y = out_ref[pl.ds(c * CH, CH)] * s
            out_ref[pl.ds(c * CH, CH)] = y * jax.nn.sigmoid(y)
            return _
        lax.fori_loop(0, N_DEV, epi_step, None)

    return pl.pallas_call(
        body,
        out_shape=jax.ShapeDtypeStruct((M, N), jnp.float32),
        in_specs=[
            pl.BlockSpec(memory_space=pltpu.VMEM),
            pl.BlockSpec(memory_space=pltpu.VMEM),
            pl.BlockSpec(memory_space=pltpu.SMEM),
            pl.BlockSpec(memory_space=pltpu.SMEM),
        ],
        out_specs=pl.BlockSpec(memory_space=pltpu.VMEM),
        scratch_shapes=[
            pltpu.VMEM((2, CH, HN), jnp.bfloat16),
            pltpu.VMEM((2, CH, HN), jnp.bfloat16),
            pltpu.SemaphoreType.DMA((2,)),
            pltpu.SemaphoreType.DMA((2,)),
            pltpu.SemaphoreType.REGULAR,
            pltpu.VMEM((2, CH, HN), jnp.bfloat16),
            pltpu.VMEM((2, CH, HN), jnp.bfloat16),
            pltpu.SemaphoreType.DMA((2,)),
            pltpu.SemaphoreType.DMA((2,)),
            pltpu.SemaphoreType.REGULAR,
        ],
        compiler_params=pltpu.CompilerParams(
            vmem_limit_bytes=56 * 1024 * 1024),
    )(x, w_mat, scale_x, scale_w)


# device time: 328820 ns/iter; 2.4268x vs baseline; 1.0135x over previous
import jax
import jax.numpy as jnp
from jax import lax
from jax.experimental import pallas as pl
from jax.experimental.pallas import tpu as pltpu

N_DEV = 16


def kernel(x, w_mat, scale_x, scale_w):
    M, K = x.shape
    N = w_mat.shape[1]
    CH = M // N_DEV
    HN = N // 2
    HOPS = N_DEV - 1

    def body(x_ref, w_ref, sx_ref, sw_ref, out_ref, acc,
             commR, sendR, recvR, creditR,
             commL, sendL, recvL, creditL):
        p = lax.axis_index("i")
        left = lax.rem(p + N_DEV - 1, N_DEV)
        right = lax.rem(p + 1, N_DEV)

        def row(c):
            return pl.ds(lax.rem(c + 2 * N_DEV, N_DEV) * CH, CH)

        colR = pl.ds(0, HN)
        colL = pl.ds(HN, HN)

        w = w_ref[...].astype(jnp.bfloat16)

        def gemm_step(c, _):
            acc[pl.ds(c * CH, CH)] = jnp.dot(
                x_ref[pl.ds(c * CH, CH)].astype(jnp.bfloat16), w,
                preferred_element_type=jnp.float32).astype(jnp.bfloat16)
            return _
        lax.fori_loop(0, N_DEV, gemm_step, None)

        def exchange(g, srcR_c, srcL_c, procR, procL):
            slot = lax.rem(g, 2)

            @pl.when(g >= 2)
            def _():
                pl.semaphore_wait(creditR, 1)
                pl.semaphore_wait(creditL, 1)

            rdmaR = pltpu.make_async_remote_copy(
                src_ref=acc.at[row(srcR_c), colR], dst_ref=commR.at[slot],
                send_sem=sendR.at[slot], recv_sem=recvR.at[slot],
                device_id=(right,), device_id_type=pl.DeviceIdType.MESH)
            rdmaL = pltpu.make_async_remote_copy(
                src_ref=acc.at[row(srcL_c), colL], dst_ref=commL.at[slot],
                send_sem=sendL.at[slot], recv_sem=recvL.at[slot],
                device_id=(left,), device_id_type=pl.DeviceIdType.MESH)
            rdmaR.start()
            rdmaL.start()
            rdmaR.wait()
            procR(slot)
            rdmaL.wait()
            procL(slot)

        def credits(g):
            @pl.when(g <= 2 * HOPS - 3)
            def _():
                pl.semaphore_signal(creditR, inc=1, device_id=(left,),
                                    device_id_type=pl.DeviceIdType.MESH)
                pl.semaphore_signal(creditL, inc=1, device_id=(right,),
                                    device_id_type=pl.DeviceIdType.MESH)

        def rs_step(t, _):
            def addR(slot):
                r = row(p - t - 1)
                acc[r, colR] = (acc[r, colR].astype(jnp.float32)
                                + commR[slot].astype(jnp.float32)
                                ).astype(jnp.bfloat16)

            def addL(slot):
                r = row(p + t + 1)
                acc[r, colL] = (acc[r, colL].astype(jnp.float32)
                                + commL[slot].astype(jnp.float32)
                                ).astype(jnp.bfloat16)
            exchange(t, p - t, p + t, addR, addL)
            credits(t)
            return _
        lax.fori_loop(0, HOPS, rs_step, None)

        def ag_step(t, _):
            g = HOPS + t

            def cpR(slot):
                acc[row(p - t), colR] = commR[slot]

            def cpL(slot):
                acc[row(p + t), colL] = commL[slot]
            exchange(g, p + 1 - t, p - 1 + t, cpR, cpL)
            credits(g)
            return _
        lax.fori_loop(0, HOPS, ag_step, None)

        s = sx_ref[0] * sw_ref[0]

        def epi_step(c, _):
            y = acc[pl.ds(c * CH, CH)].astype(jnp.float32) * s
            out_ref[pl.ds(c * CH, CH)] = y * jax.nn.sigmoid(y)
            return _
        lax.fori_loop(0, N_DEV, epi_step, None)

    return pl.pallas_call(
        body,
        out_shape=jax.ShapeDtypeStruct((M, N), jnp.float32),
        in_specs=[
            pl.BlockSpec(memory_space=pltpu.VMEM),
            pl.BlockSpec(memory_space=pltpu.VMEM),
            pl.BlockSpec(memory_space=pltpu.SMEM),
            pl.BlockSpec(memory_space=pltpu.SMEM),
        ],
        out_specs=pl.BlockSpec(memory_space=pltpu.VMEM),
        scratch_shapes=[
            pltpu.VMEM((M, N), jnp.bfloat16),
            pltpu.VMEM((2, CH, HN), jnp.bfloat16),
            pltpu.SemaphoreType.DMA((2,)),
            pltpu.SemaphoreType.DMA((2,)),
            pltpu.SemaphoreType.REGULAR,
            pltpu.VMEM((2, CH, HN), jnp.bfloat16),
            pltpu.SemaphoreType.DMA((2,)),
            pltpu.SemaphoreType.DMA((2,)),
            pltpu.SemaphoreType.REGULAR,
        ],
        compiler_params=pltpu.CompilerParams(
            vmem_limit_bytes=58 * 1024 * 1024),
    )(x, w_mat, scale_x, scale_w)


# device time: 243992 ns/iter; 3.2705x vs baseline; 1.3477x over previous
import jax
import jax.numpy as jnp
from jax import lax
from jax.experimental import pallas as pl
from jax.experimental.pallas import tpu as pltpu

N_DEV = 16
NSUB = 2


def kernel(x, w_mat, scale_x, scale_w):
    M, K = x.shape
    N = w_mat.shape[1]
    CH = M // N_DEV
    HN = N // 2
    QW = HN // NSUB
    HOPS = N_DEV - 1
    G = 2 * HOPS

    RINGS = [(+1, s * QW) for s in range(NSUB)] + \
            [(-1, HN + s * QW) for s in range(NSUB)]

    def body(x_ref, w_ref, sx_ref, sw_ref, out_ref, acc, *scr):
        p = lax.axis_index("i")
        left = lax.rem(p + N_DEV - 1, N_DEV)
        right = lax.rem(p + 1, N_DEV)

        rings = []
        for i, (d, col) in enumerate(RINGS):
            comm, send, recv, credit = scr[4 * i:4 * i + 4]
            rings.append(dict(
                d=d, col=col, comm=comm, send=send, recv=recv,
                credit=credit,
                tgt=right if d == 1 else left,
                cred_tgt=left if d == 1 else right,
            ))

        def row(c):
            return pl.ds(lax.rem(c + 4 * N_DEV, N_DEV) * CH, CH)

        w = w_ref[...].astype(jnp.bfloat16)

        def gemm_step(c, _):
            acc[pl.ds(c * CH, CH)] = jnp.dot(
                x_ref[pl.ds(c * CH, CH)].astype(jnp.bfloat16), w,
                preferred_element_type=jnp.float32).astype(jnp.bfloat16)
            return _
        lax.fori_loop(0, N_DEV, gemm_step, None)

        def desc(r, slot, src_c):
            return pltpu.make_async_remote_copy(
                src_ref=acc.at[row(src_c), pl.ds(r["col"], QW)],
                dst_ref=r["comm"].at[slot],
                send_sem=r["send"].at[slot], recv_sem=r["recv"].at[slot],
                device_id=(r["tgt"],), device_id_type=pl.DeviceIdType.MESH)

        def send_chunk(r, g):
            return lax.select(g < HOPS,
                              p - r["d"] * g,
                              p + r["d"] * (1 - (g - HOPS)))

        def process(r, h):
            slot = lax.rem(h, 2)
            cs = pl.ds(r["col"], QW)

            @pl.when(h < HOPS)
            def _():
                rr = row(p - r["d"] * (h + 1))
                acc[rr, cs] = (acc[rr, cs].astype(jnp.float32)
                               + r["comm"][slot].astype(jnp.float32)
                               ).astype(jnp.bfloat16)

            @pl.when(h >= HOPS)
            def _():
                acc[row(p - r["d"] * (h - HOPS)), cs] = r["comm"][slot]

            @pl.when(h <= G - 3)
            def _():
                pl.semaphore_signal(
                    r["credit"], inc=1, device_id=(r["cred_tgt"],),
                    device_id_type=pl.DeviceIdType.MESH)

        for r in rings:
            desc(r, 0, p - r["d"] * 0).start()

        def step(g, _):
            for r in rings:
                desc(r, lax.rem(g - 1, 2), send_chunk(r, g - 1)).wait()
                process(r, g - 1)

                @pl.when(g >= 2)
                def _():
                    pl.semaphore_wait(r["credit"], 1)
                desc(r, lax.rem(g, 2), send_chunk(r, g)).start()
            return _
        lax.fori_loop(1, G, step, None)

        for r in rings:
            desc(r, (G - 1) % 2, send_chunk(r, G - 1)).wait()
            process(r, G - 1)

        s = sx_ref[0] * sw_ref[0]

        def epi_step(c, _):
            y = acc[pl.ds(c * CH, CH)].astype(jnp.float32) * s
            out_ref[pl.ds(c * CH, CH)] = y * jax.nn.sigmoid(y)
            return _
        lax.fori_loop(0, N_DEV, epi_step, None)

    ring_scratch = []
    for _ in RINGS:
        ring_scratch += [
            pltpu.VMEM((2, CH, QW), jnp.bfloat16),
            pltpu.SemaphoreType.DMA((2,)),
            pltpu.SemaphoreType.DMA((2,)),
            pltpu.SemaphoreType.REGULAR,
        ]

    return pl.pallas_call(
        body,
        out_shape=jax.ShapeDtypeStruct((M, N), jnp.float32),
        in_specs=[
            pl.BlockSpec(memory_space=pltpu.VMEM),
            pl.BlockSpec(memory_space=pltpu.VMEM),
            pl.BlockSpec(memory_space=pltpu.SMEM),
            pl.BlockSpec(memory_space=pltpu.SMEM),
        ],
        out_specs=pl.BlockSpec(memory_space=pltpu.VMEM),
        scratch_shapes=[pltpu.VMEM((M, N), jnp.bfloat16)] + ring_scratch,
        compiler_params=pltpu.CompilerParams(
            vmem_limit_bytes=58 * 1024 * 1024),
    )(x, w_mat, scale_x, scale_w)
